# Initial kernel scaffold; baseline (speedup 1.0000x reference)
import jax
import jax.numpy as jnp
from jax.experimental import pallas as pl


def kernel(sample, graph_batch, p_feature, desc_table, tweet_table, W_t, b_t, W1, b1, gamma, beta, W2, b2):
    raise NotImplementedError("write your pallas kernel here")


# speedup vs baseline: 7.8713x; 7.8713x over previous
import functools

import jax
import jax.numpy as jnp
from jax import lax
from jax.experimental import pallas as pl
from jax.experimental.pallas import tpu as pltpu
from jax.experimental.pallas import tpu_sc as plsc

NUM_NODES = 65536
NHID = 128
EDIM = 1024
B = 128


def _sc_gather_kernel(desc_idx_hbm, tweet_idx_hbm, gb_hbm, p_hbm, desc_hbm,
                      tweet_hbm, e_out, pg_out,
                      probe_v, prows_v, didx_v, drows_v, tidx_v, trows_v,
                      sem_a, sem_b):
    cid = lax.axis_index("c")
    sid = lax.axis_index("s")
    wid = sid * 2 + cid

    @pl.when(wid >= 16)
    def _desc_gather():
        j = wid - 16
        pltpu.sync_copy(desc_idx_hbm.at[j], didx_v)
        pltpu.async_copy(desc_hbm.at[didx_v], drows_v, sem_a).wait()
        pltpu.sync_copy(drows_v, e_out.at[pl.ds(j * 16, 16)])

    @pl.when(wid < 16)
    def _search_and_gather():
        pltpu.sync_copy(tweet_idx_hbm.at[wid], tidx_v)
        tweet_copy = pltpu.make_async_copy(tweet_hbm.at[tidx_v], trows_v,
                                           sem_b)
        tweet_copy.start()

        base_id = (wid & 7) * 16
        add_one = wid >> 3
        b_ids = base_id + lax.broadcasted_iota(jnp.int32, (16,), 0)
        lo0 = jnp.zeros((16,), jnp.int32)
        hi0 = jnp.full((16,), NUM_NODES, jnp.int32)

        def body(_, carry):
            lo, hi = carry
            mid = (lo + hi) >> 1
            pltpu.async_copy(gb_hbm.at[mid], probe_v, sem_a).wait()
            pred = probe_v[...] < b_ids
            return jnp.where(pred, mid + 1, lo), jnp.where(pred, hi, mid)

        lo, _ = lax.fori_loop(0, 16, body, (lo0, hi0))
        idx16 = jnp.minimum(lo + add_one, NUM_NODES - 1)

        pltpu.async_copy(p_hbm.at[idx16], prows_v, sem_a).wait()
        pltpu.sync_copy(prows_v, pg_out.at[pl.ds(wid * 16, 16)])

        tweet_copy.wait()
        pltpu.sync_copy(trows_v, e_out.at[pl.ds(256 + wid * 8, 8)])


def _sc_gather(desc_idx, tweet_idx, graph_batch, p_feature, desc_table,
               tweet_table):
    mesh = plsc.VectorSubcoreMesh(core_axis_name="c", subcore_axis_name="s")
    f = pl.kernel(
        _sc_gather_kernel,
        mesh=mesh,
        out_type=[
            jax.ShapeDtypeStruct((3 * B, EDIM), jnp.float32),
            jax.ShapeDtypeStruct((2 * B, NHID), jnp.float32),
        ],
        scratch_types=[
            pltpu.VMEM((16,), jnp.int32),
            pltpu.VMEM((16, NHID), jnp.float32),
            pltpu.VMEM((16,), jnp.int32),
            pltpu.VMEM((16, EDIM), jnp.float32),
            pltpu.VMEM((8,), jnp.int32),
            pltpu.VMEM((8, EDIM), jnp.float32),
            pltpu.SemaphoreType.DMA,
            pltpu.SemaphoreType.DMA,
        ],
    )
    return f(desc_idx, tweet_idx, graph_batch, p_feature, desc_table,
             tweet_table)


def _dense_body(e_ref, pg_ref, wt_ref, bt_ref, w1_ref, b1_ref, g_ref, be_ref,
                w2_ref, b2_ref, out_ref):
    t = jnp.dot(e_ref[...], wt_ref[...],
                preferred_element_type=jnp.float32) + bt_ref[...]
    t_d0 = t[0:128]
    t_d2 = t[128:256]
    t_tw = t[256:384]
    u = pg_ref[0:128]
    v = pg_ref[128:256]
    x = jnp.concatenate([u + t_d0, t_tw, v + t_d2], axis=1)
    h = jnp.dot(x, w1_ref[...],
                preferred_element_type=jnp.float32) + b1_ref[...]
    mean = jnp.mean(h, axis=0, keepdims=True)
    var = jnp.mean(jnp.square(h - mean), axis=0, keepdims=True)
    hn = (h - mean) * lax.rsqrt(var + 1e-5) * g_ref[...] + be_ref[...]
    hr = jnp.maximum(hn, 0.0)
    o = jnp.sum(hr * w2_ref[...], axis=1, keepdims=True) + b2_ref[...]
    out_ref[...] = 1.0 / (1.0 + jnp.exp(-o))


def _dense(e, pg, w_t, b_t, w1, b1, gamma, beta, w2, b2):
    return pl.pallas_call(
        _dense_body,
        out_shape=jax.ShapeDtypeStruct((B, 1), jnp.float32),
    )(e, pg, w_t, b_t.reshape(1, NHID), w1, b1.reshape(1, 256),
      gamma.reshape(1, 256), beta.reshape(1, 256), w2.reshape(1, 256),
      b2.reshape(1, 1))


def kernel(sample, graph_batch, p_feature, desc_table, tweet_table,
           W_t, b_t, W1, b1, gamma, beta, W2, b2):
    desc_idx = jnp.concatenate([sample[:, 0], sample[:, 2]]).reshape(16, 16)
    tweet_idx = sample[:, 1].reshape(16, 8)
    e, pg = _sc_gather(desc_idx, tweet_idx, graph_batch, p_feature,
                       desc_table, tweet_table)
    return _dense(e, pg, W_t, b_t, W1, b1, gamma, beta, W2, b2)
